# Initial kernel scaffold; baseline (speedup 1.0000x reference)
import jax
import jax.numpy as jnp
from jax.experimental import pallas as pl


def kernel(x, edge_index, W1, b1, W2, b2):
    raise NotImplementedError("write your pallas kernel here")


# speedup vs baseline: 51.2724x; 51.2724x over previous
import functools

import jax
import jax.numpy as jnp
from jax import lax
from jax.experimental import pallas as pl
from jax.experimental.pallas import tpu as pltpu
from jax.experimental.pallas import tpu_sc as plsc

_N = 10000
_E = 320000
_NFEAT = 128
_NHID = 128
_NCLASS = 64
_K = 10
_ALPHA = 0.1

_NT = 16
_NC = 2
_F = _NCLASS // _NC
_RPT = 640
_NPAD = _NT * _RPT
_CH = 128
_NCHUNK = 160
_EPT = _CH * _NCHUNK
_EPAD = _NT * _EPT
_RCH = _RPT // _CH


def _mlp_body(x_ref, w1_ref, b1_ref, w2_ref, b2_ref, o_ref):
    h = jnp.dot(x_ref[...], w1_ref[...], preferred_element_type=jnp.float32)
    h = jnp.maximum(h + b1_ref[...], 0.0)
    o_ref[...] = jnp.dot(h, w2_ref[...], preferred_element_type=jnp.float32) + b2_ref[...]


def _mlp(x, W1, b1, W2, b2):
    blk = 1024
    return pl.pallas_call(
        _mlp_body,
        grid=(_NPAD // blk,),
        in_specs=[
            pl.BlockSpec((blk, _NFEAT), lambda i: (i, 0)),
            pl.BlockSpec((_NFEAT, _NHID), lambda i: (0, 0)),
            pl.BlockSpec((1, _NHID), lambda i: (0, 0)),
            pl.BlockSpec((_NHID, _NCLASS), lambda i: (0, 0)),
            pl.BlockSpec((1, _NCLASS), lambda i: (0, 0)),
        ],
        out_specs=pl.BlockSpec((blk, _NCLASS), lambda i: (i, 0)),
        out_shape=jax.ShapeDtypeStruct((_NPAD, _NCLASS), jnp.float32),
    )(x, W1, b1.reshape(1, _NHID), W2, b2.reshape(1, _NCLASS))


def _lsm_body(z_ref, o_ref):
    z = z_ref[...]
    m = jnp.max(z, axis=1, keepdims=True)
    e = jnp.exp(z - m)
    s = jnp.sum(e, axis=1, keepdims=True)
    o_ref[...] = z - m - jnp.log(s)


def _lsm(z):
    blk = 1000
    return pl.pallas_call(
        _lsm_body,
        grid=(_N // blk,),
        in_specs=[pl.BlockSpec((blk, _NCLASS), lambda i: (i, 0))],
        out_specs=pl.BlockSpec((blk, _NCLASS), lambda i: (i, 0)),
        out_shape=jax.ShapeDtypeStruct((_N, _NCLASS), jnp.float32),
    )(z)


def _make_prop():
    mesh = plsc.VectorSubcoreMesh(
        core_axis_name="c", subcore_axis_name="s",
        num_cores=_NC, num_subcores=_NT)

    @functools.partial(
        pl.kernel,
        out_type=jax.ShapeDtypeStruct((_NPAD, _NCLASS), jnp.float32),
        mesh=mesh,
        compiler_params=pltpu.CompilerParams(
            needs_layout_passes=False, use_tc_tiling_on_sc=False),
        scratch_types=[
            pltpu.HBM((_NC, _NPAD, _F), jnp.float32),
            pltpu.VMEM_SHARED((_NPAD, _F), jnp.float32),
            pltpu.VMEM_SHARED((_NPAD,), jnp.float32),
            pltpu.VMEM((_NCHUNK, _CH), jnp.int32),
            pltpu.VMEM((_NCHUNK, _CH), jnp.int32),
            pltpu.VMEM((_RPT, _F), jnp.float32),
            pltpu.VMEM((_RPT,), jnp.float32),
            pltpu.VMEM((_CH,), jnp.float32),
            pltpu.VMEM((10, _CH, _F), jnp.float32),
            pltpu.VMEM((_CH, _F), jnp.float32),
            pltpu.VMEM((_RPT,), jnp.float32),
            pltpu.SemaphoreType.DMA,
            pltpu.SemaphoreType.DMA,
        ],
    )
    def prop(z0_ref, edges_ref, out_ref,
             Zfull, S, deg_sh, src_v, dst_v, h0_v, degsl_v, ones_v,
             gbuf, ubuf, dinv_v, gsem, ssem):
        c = lax.axis_index("c")
        t = lax.axis_index("s")
        rbase = t * _RPT
        fbase = c * _F
        Z = Zfull.at[c]

        pltpu.sync_copy(edges_ref.at[0, t], src_v)
        pltpu.sync_copy(edges_ref.at[1, t], dst_v)
        pltpu.sync_copy(
            z0_ref.at[pl.ds(rbase, _RPT), pl.ds(fbase, _F)], h0_v)

        def _zero(i, u):
            degsl_v[pl.ds(i * 16, 16)] = jnp.zeros((16,), jnp.float32)
            return u
        lax.fori_loop(0, _RPT // 16, _zero, 0)
        for g in range(_CH // 16):
            ones_v[pl.ds(g * 16, 16)] = jnp.ones((16,), jnp.float32)
        pltpu.sync_copy(degsl_v, deg_sh.at[pl.ds(rbase, _RPT)])
        plsc.subcore_barrier()

        def _deg(j, u):
            pltpu.sync_copy(ones_v, deg_sh.at[dst_v.at[j]], add=True)
            return u
        lax.fori_loop(0, _NCHUNK, _deg, 0)
        plsc.subcore_barrier()

        pltpu.sync_copy(deg_sh.at[pl.ds(rbase, _RPT)], degsl_v)

        def _dinv(i, u):
            d = degsl_v[pl.ds(i * 16, 16)] + 1.0
            bits = plsc.bitcast(d, jnp.int32)
            y = plsc.bitcast(jnp.int32(0x5F3759DF) - (bits >> 1), jnp.float32)
            for _ in range(3):
                y = y * (1.5 - 0.5 * d * y * y)
            dinv_v[pl.ds(i * 16, 16)] = y
            return u
        lax.fori_loop(0, _RPT // 16, _dinv, 0)

        for ch in range(_RCH):
            def _initrow(g16, u, ch=ch):
                dv = dinv_v[pl.ds(ch * _CH + g16 * 16, 16)]
                for r in range(16):
                    row = g16 * 16 + r
                    av = lax.broadcast(dv[r], (16,))
                    for g in range(_F // 16):
                        sl = pl.ds(g * 16, 16)
                        ubuf[row, sl] = av * h0_v[ch * _CH + row, sl]
                return u
            lax.fori_loop(0, _CH // 16, _initrow, 0)
            rows = pl.ds(rbase + ch * _CH, _CH)
            pltpu.sync_copy(ubuf, Z.at[rows])
            pltpu.sync_copy(ubuf, S.at[rows])

        def _edge_pass():
            plsc.subcore_barrier()
            for p in range(8):
                pltpu.async_copy(Z.at[src_v.at[p]], gbuf.at[p], gsem)

            def _eb(jj, u):
                for p in range(10):
                    j = jj * 10 + p
                    pltpu.make_async_copy(
                        Z.at[src_v.at[j]], gbuf.at[p], gsem).wait()
                    pltpu.async_copy(gbuf.at[p], S.at[dst_v.at[j]], ssem, add=True)
                    @pl.when(j + 8 < _NCHUNK)
                    def _():
                        @pl.when(j >= 2)
                        def _():
                            pltpu.make_async_copy(
                                gbuf.at[(p + 8) % 10],
                                S.at[dst_v.at[j - 2]], ssem).wait()
                        pltpu.async_copy(
                            Z.at[src_v.at[j + 8]], gbuf.at[(p + 8) % 10], gsem)
                return u
            lax.fori_loop(0, _NCHUNK // 10, _eb, 0)
            for p in range(10):
                pltpu.make_async_copy(
                    gbuf.at[(_NCHUNK - 10 + p) % 10],
                    S.at[dst_v.at[_NCHUNK - 10 + p]], ssem).wait()
            plsc.subcore_barrier()

        def _update(write_out):
            for ch in range(_RCH):
                rows = pl.ds(rbase + ch * _CH, _CH)
                pltpu.sync_copy(S.at[rows], gbuf.at[0])

                def _ur(g16, u, ch=ch):
                    dv = dinv_v[pl.ds(ch * _CH + g16 * 16, 16)]
                    for r in range(16):
                        row = g16 * 16 + r
                        av = lax.broadcast(dv[r], (16,))
                        for g in range(_F // 16):
                            sl = pl.ds(g * 16, 16)
                            z = ((1.0 - _ALPHA) * (av * gbuf[0, row, sl])
                                 + _ALPHA * h0_v[ch * _CH + row, sl])
                            ubuf[row, sl] = z if write_out else av * z
                    return u
                lax.fori_loop(0, _CH // 16, _ur, 0)
                if write_out:
                    pltpu.sync_copy(
                        ubuf, out_ref.at[rows, pl.ds(fbase, _F)])
                else:
                    pltpu.sync_copy(ubuf, Z.at[rows])
                    pltpu.sync_copy(ubuf, S.at[rows])

        def _ib(k, u):
            _edge_pass()
            _update(False)
            return u
        lax.fori_loop(0, _K - 1, _ib, 0)
        _edge_pass()
        _update(True)

    return prop


_prop = _make_prop()


def kernel(x, edge_index, W1, b1, W2, b2):
    z0 = _mlp(x, W1, b1, W2, b2)

    pad_e = _EPAD - _E
    pad_src = (jnp.arange(pad_e, dtype=jnp.int32) * 37) % _N
    pad_dst = _N + (jnp.arange(pad_e, dtype=jnp.int32) % (_NPAD - _N))
    pad_pair = jnp.stack([pad_src, pad_dst])
    edges = jnp.concatenate([edge_index, pad_pair], axis=1)
    edges = edges.reshape(2, _NT, _NCHUNK, _CH)

    zfin = _prop(z0, edges)
    return _lsm(zfin)


# speedup vs baseline: 52.2547x; 1.0192x over previous
import functools

import jax
import jax.numpy as jnp
from jax import lax
from jax.experimental import pallas as pl
from jax.experimental.pallas import tpu as pltpu
from jax.experimental.pallas import tpu_sc as plsc

_N = 10000
_E = 320000
_NFEAT = 128
_NHID = 128
_NCLASS = 64
_K = 10
_ALPHA = 0.1

_NT = 16
_NC = 2
_F = _NCLASS // _NC
_RPT = 640
_NPAD = _NT * _RPT
_CH = 128
_NCHUNK = 160
_EPT = _CH * _NCHUNK
_EPAD = _NT * _EPT
_RCH = _RPT // _CH


def _mlp_body(x_ref, w1_ref, b1_ref, w2_ref, b2_ref, o_ref):
    h = jnp.dot(x_ref[...], w1_ref[...], preferred_element_type=jnp.float32)
    h = jnp.maximum(h + b1_ref[...], 0.0)
    o_ref[...] = jnp.dot(h, w2_ref[...], preferred_element_type=jnp.float32) + b2_ref[...]


def _mlp(x, W1, b1, W2, b2):
    blk = 1024
    return pl.pallas_call(
        _mlp_body,
        grid=(_NPAD // blk,),
        in_specs=[
            pl.BlockSpec((blk, _NFEAT), lambda i: (i, 0)),
            pl.BlockSpec((_NFEAT, _NHID), lambda i: (0, 0)),
            pl.BlockSpec((1, _NHID), lambda i: (0, 0)),
            pl.BlockSpec((_NHID, _NCLASS), lambda i: (0, 0)),
            pl.BlockSpec((1, _NCLASS), lambda i: (0, 0)),
        ],
        out_specs=pl.BlockSpec((blk, _NCLASS), lambda i: (i, 0)),
        out_shape=jax.ShapeDtypeStruct((_NPAD, _NCLASS), jnp.float32),
    )(x, W1, b1.reshape(1, _NHID), W2, b2.reshape(1, _NCLASS))


def _lsm_body(z_ref, o_ref):
    z = z_ref[...]
    m = jnp.max(z, axis=1, keepdims=True)
    e = jnp.exp(z - m)
    s = jnp.sum(e, axis=1, keepdims=True)
    o_ref[...] = z - m - jnp.log(s)


def _lsm(z):
    blk = 1000
    return pl.pallas_call(
        _lsm_body,
        grid=(_N // blk,),
        in_specs=[pl.BlockSpec((blk, _NCLASS), lambda i: (i, 0))],
        out_specs=pl.BlockSpec((blk, _NCLASS), lambda i: (i, 0)),
        out_shape=jax.ShapeDtypeStruct((_N, _NCLASS), jnp.float32),
    )(z)


def _make_prop():
    mesh = plsc.VectorSubcoreMesh(
        core_axis_name="c", subcore_axis_name="s",
        num_cores=_NC, num_subcores=_NT)

    @functools.partial(
        pl.kernel,
        out_type=jax.ShapeDtypeStruct((_NPAD, _NCLASS), jnp.float32),
        mesh=mesh,
        compiler_params=pltpu.CompilerParams(
            needs_layout_passes=False, use_tc_tiling_on_sc=False),
        scratch_types=[
            pltpu.HBM((_NC, _NPAD, _F), jnp.float32),
            pltpu.VMEM_SHARED((_NPAD, _F), jnp.float32),
            pltpu.VMEM_SHARED((_NPAD,), jnp.float32),
            pltpu.VMEM((_NCHUNK, _CH), jnp.int32),
            pltpu.VMEM((_NCHUNK, _CH), jnp.int32),
            pltpu.VMEM((_RPT, _F), jnp.float32),
            pltpu.VMEM((_RPT,), jnp.float32),
            pltpu.VMEM((_CH,), jnp.float32),
            pltpu.VMEM((10, _CH, _F), jnp.float32),
            pltpu.VMEM((_CH, _F), jnp.float32),
            pltpu.VMEM((_RPT,), jnp.float32),
            pltpu.SemaphoreType.DMA,
            pltpu.SemaphoreType.DMA,
        ],
    )
    def prop(z0_ref, edges_ref, out_ref,
             Zfull, S, deg_sh, src_v, dst_v, h0_v, degsl_v, ones_v,
             gbuf, ubuf, dinv_v, gsem, ssem):
        c = lax.axis_index("c")
        t = lax.axis_index("s")
        rbase = t * _RPT
        fbase = c * _F
        Z = Zfull.at[c]

        pltpu.sync_copy(edges_ref.at[0, t], src_v)
        pltpu.sync_copy(edges_ref.at[1, t], dst_v)
        pltpu.sync_copy(
            z0_ref.at[pl.ds(rbase, _RPT), pl.ds(fbase, _F)], h0_v)

        def _zero(i, u):
            degsl_v[pl.ds(i * 16, 16)] = jnp.zeros((16,), jnp.float32)
            return u
        lax.fori_loop(0, _RPT // 16, _zero, 0)
        for g in range(_CH // 16):
            ones_v[pl.ds(g * 16, 16)] = jnp.ones((16,), jnp.float32)
        pltpu.sync_copy(degsl_v, deg_sh.at[pl.ds(rbase, _RPT)])
        plsc.subcore_barrier()

        for p in range(8):
            pltpu.async_copy(ones_v, deg_sh.at[dst_v.at[p]], ssem, add=True)

        def _deg(j, u):
            pltpu.make_async_copy(ones_v, deg_sh.at[dst_v.at[j]], ssem).wait()

            @pl.when(j + 8 < _NCHUNK)
            def _():
                pltpu.async_copy(
                    ones_v, deg_sh.at[dst_v.at[j + 8]], ssem, add=True)
            return u
        lax.fori_loop(0, _NCHUNK, _deg, 0)
        plsc.subcore_barrier()

        pltpu.sync_copy(deg_sh.at[pl.ds(rbase, _RPT)], degsl_v)

        def _dinv(i, u):
            d = degsl_v[pl.ds(i * 16, 16)] + 1.0
            bits = plsc.bitcast(d, jnp.int32)
            y = plsc.bitcast(jnp.int32(0x5F3759DF) - (bits >> 1), jnp.float32)
            for _ in range(3):
                y = y * (1.5 - 0.5 * d * y * y)
            dinv_v[pl.ds(i * 16, 16)] = y
            return u
        lax.fori_loop(0, _RPT // 16, _dinv, 0)

        for ch in range(_RCH):
            def _initrow(g16, u, ch=ch):
                dv = dinv_v[pl.ds(ch * _CH + g16 * 16, 16)]
                for r in range(16):
                    row = g16 * 16 + r
                    av = lax.broadcast(dv[r], (16,))
                    for g in range(_F // 16):
                        sl = pl.ds(g * 16, 16)
                        ubuf[row, sl] = av * h0_v[ch * _CH + row, sl]
                return u
            lax.fori_loop(0, _CH // 16, _initrow, 0)
            rows = pl.ds(rbase + ch * _CH, _CH)
            pltpu.sync_copy(ubuf, Z.at[rows])
            pltpu.sync_copy(ubuf, S.at[rows])

        def _edge_pass():
            plsc.subcore_barrier()
            for p in range(8):
                pltpu.async_copy(Z.at[src_v.at[p]], gbuf.at[p], gsem)

            def _eb(jj, u):
                for p in range(10):
                    j = jj * 10 + p
                    pltpu.make_async_copy(
                        Z.at[src_v.at[j]], gbuf.at[p], gsem).wait()
                    pltpu.async_copy(gbuf.at[p], S.at[dst_v.at[j]], ssem, add=True)
                    @pl.when(j + 8 < _NCHUNK)
                    def _():
                        @pl.when(j >= 2)
                        def _():
                            pltpu.make_async_copy(
                                gbuf.at[(p + 8) % 10],
                                S.at[dst_v.at[j - 2]], ssem).wait()
                        pltpu.async_copy(
                            Z.at[src_v.at[j + 8]], gbuf.at[(p + 8) % 10], gsem)
                return u
            lax.fori_loop(0, _NCHUNK // 10, _eb, 0)
            for p in range(10):
                pltpu.make_async_copy(
                    gbuf.at[(_NCHUNK - 10 + p) % 10],
                    S.at[dst_v.at[_NCHUNK - 10 + p]], ssem).wait()
            plsc.subcore_barrier()

        def _update(write_out):
            for ch in range(_RCH):
                rows = pl.ds(rbase + ch * _CH, _CH)
                pltpu.sync_copy(S.at[rows], gbuf.at[0])

                def _ur(g16, u, ch=ch):
                    dv = dinv_v[pl.ds(ch * _CH + g16 * 16, 16)]
                    for r in range(16):
                        row = g16 * 16 + r
                        av = lax.broadcast(dv[r], (16,))
                        for g in range(_F // 16):
                            sl = pl.ds(g * 16, 16)
                            z = ((1.0 - _ALPHA) * (av * gbuf[0, row, sl])
                                 + _ALPHA * h0_v[ch * _CH + row, sl])
                            ubuf[row, sl] = z if write_out else av * z
                    return u
                lax.fori_loop(0, _CH // 16, _ur, 0)
                if write_out:
                    pltpu.sync_copy(
                        ubuf, out_ref.at[rows, pl.ds(fbase, _F)])
                else:
                    pltpu.sync_copy(ubuf, Z.at[rows])
                    pltpu.sync_copy(ubuf, S.at[rows])

        def _ib(k, u):
            _edge_pass()
            _update(False)
            return u
        lax.fori_loop(0, _K - 1, _ib, 0)
        _edge_pass()
        _update(True)

    return prop


_prop = _make_prop()


def kernel(x, edge_index, W1, b1, W2, b2):
    z0 = _mlp(x, W1, b1, W2, b2)

    pad_e = _EPAD - _E
    pad_src = (jnp.arange(pad_e, dtype=jnp.int32) * 37) % _N
    pad_dst = _N + (jnp.arange(pad_e, dtype=jnp.int32) % (_NPAD - _N))
    pad_pair = jnp.stack([pad_src, pad_dst])
    edges = jnp.concatenate([edge_index, pad_pair], axis=1)
    edges = edges.reshape(2, _NT, _NCHUNK, _CH)

    zfin = _prop(z0, edges)
    return _lsm(zfin)


# speedup vs baseline: 53.1568x; 1.0173x over previous
import functools

import jax
import jax.numpy as jnp
from jax import lax
from jax.experimental import pallas as pl
from jax.experimental.pallas import tpu as pltpu
from jax.experimental.pallas import tpu_sc as plsc

_N = 10000
_E = 320000
_NFEAT = 128
_NHID = 128
_NCLASS = 64
_K = 10
_ALPHA = 0.1

_NT = 16
_NC = 2
_F = _NCLASS // _NC
_RPT = 640
_NPAD = _NT * _RPT
_CH = 128
_NCHUNK = 160
_EPT = _CH * _NCHUNK
_EPAD = _NT * _EPT
_RCH = _RPT // _CH


def _mlp_body(x_ref, w1_ref, b1_ref, w2_ref, b2_ref, o_ref):
    h = jnp.dot(x_ref[...], w1_ref[...], preferred_element_type=jnp.float32)
    h = jnp.maximum(h + b1_ref[...], 0.0)
    o_ref[...] = jnp.dot(h, w2_ref[...], preferred_element_type=jnp.float32) + b2_ref[...]


def _mlp(x, W1, b1, W2, b2):
    blk = 1024
    return pl.pallas_call(
        _mlp_body,
        grid=(_NPAD // blk,),
        in_specs=[
            pl.BlockSpec((blk, _NFEAT), lambda i: (i, 0)),
            pl.BlockSpec((_NFEAT, _NHID), lambda i: (0, 0)),
            pl.BlockSpec((1, _NHID), lambda i: (0, 0)),
            pl.BlockSpec((_NHID, _NCLASS), lambda i: (0, 0)),
            pl.BlockSpec((1, _NCLASS), lambda i: (0, 0)),
        ],
        out_specs=pl.BlockSpec((blk, _NCLASS), lambda i: (i, 0)),
        out_shape=jax.ShapeDtypeStruct((_NPAD, _NCLASS), jnp.float32),
    )(x, W1, b1.reshape(1, _NHID), W2, b2.reshape(1, _NCLASS))


def _lsm_body(z_ref, o_ref):
    z = z_ref[...]
    m = jnp.max(z, axis=1, keepdims=True)
    e = jnp.exp(z - m)
    s = jnp.sum(e, axis=1, keepdims=True)
    o_ref[...] = z - m - jnp.log(s)


def _lsm(z):
    blk = 1000
    return pl.pallas_call(
        _lsm_body,
        grid=(_N // blk,),
        in_specs=[pl.BlockSpec((blk, _NCLASS), lambda i: (i, 0))],
        out_specs=pl.BlockSpec((blk, _NCLASS), lambda i: (i, 0)),
        out_shape=jax.ShapeDtypeStruct((_N, _NCLASS), jnp.float32),
    )(z)


def _make_prop():
    mesh = plsc.VectorSubcoreMesh(
        core_axis_name="c", subcore_axis_name="s",
        num_cores=_NC, num_subcores=_NT)

    @functools.partial(
        pl.kernel,
        out_type=jax.ShapeDtypeStruct((_NPAD, _NCLASS), jnp.float32),
        mesh=mesh,
        compiler_params=pltpu.CompilerParams(
            needs_layout_passes=False, use_tc_tiling_on_sc=False),
        scratch_types=[
            pltpu.HBM((_NC, _NPAD, _F), jnp.float32),
            pltpu.VMEM_SHARED((_NPAD, _F), jnp.float32),
            pltpu.VMEM_SHARED((_NPAD,), jnp.float32),
            pltpu.VMEM((_NCHUNK, _CH), jnp.int32),
            pltpu.VMEM((_NCHUNK, _CH), jnp.int32),
            pltpu.VMEM((_RPT, _F), jnp.float32),
            pltpu.VMEM((_RPT,), jnp.float32),
            pltpu.VMEM((_CH,), jnp.float32),
            pltpu.VMEM((10, _CH, _F), jnp.float32),
            pltpu.VMEM((_CH, _F), jnp.float32),
            pltpu.VMEM((_RPT,), jnp.float32),
            pltpu.SemaphoreType.DMA,
            pltpu.SemaphoreType.DMA,
        ],
    )
    def prop(z0_ref, edges_ref, out_ref,
             Zfull, S, deg_sh, src_v, dst_v, h0_v, degsl_v, ones_v,
             gbuf, ubuf, dinv_v, gsem, ssem):
        c = lax.axis_index("c")
        t = lax.axis_index("s")
        rbase = t * _RPT
        fbase = c * _F
        Z = Zfull.at[c]

        pltpu.sync_copy(edges_ref.at[0, t], src_v)
        pltpu.sync_copy(edges_ref.at[1, t], dst_v)
        pltpu.sync_copy(
            z0_ref.at[pl.ds(rbase, _RPT), pl.ds(fbase, _F)], h0_v)

        def _zero(i, u):
            degsl_v[pl.ds(i * 16, 16)] = jnp.zeros((16,), jnp.float32)
            return u
        lax.fori_loop(0, _RPT // 16, _zero, 0)
        for g in range(_CH // 16):
            ones_v[pl.ds(g * 16, 16)] = jnp.ones((16,), jnp.float32)
        pltpu.sync_copy(degsl_v, deg_sh.at[pl.ds(rbase, _RPT)])
        plsc.subcore_barrier()

        for p in range(8):
            pltpu.async_copy(ones_v, deg_sh.at[dst_v.at[p]], ssem, add=True)

        def _deg(j, u):
            pltpu.make_async_copy(ones_v, deg_sh.at[dst_v.at[j]], ssem).wait()

            @pl.when(j + 8 < _NCHUNK)
            def _():
                pltpu.async_copy(
                    ones_v, deg_sh.at[dst_v.at[j + 8]], ssem, add=True)
            return u
        lax.fori_loop(0, _NCHUNK, _deg, 0)
        plsc.subcore_barrier()

        pltpu.sync_copy(deg_sh.at[pl.ds(rbase, _RPT)], degsl_v)

        def _dinv(i, u):
            d = degsl_v[pl.ds(i * 16, 16)] + 1.0
            bits = plsc.bitcast(d, jnp.int32)
            y = plsc.bitcast(jnp.int32(0x5F3759DF) - (bits >> 1), jnp.float32)
            for _ in range(3):
                y = y * (1.5 - 0.5 * d * y * y)
            dinv_v[pl.ds(i * 16, 16)] = y
            return u
        lax.fori_loop(0, _RPT // 16, _dinv, 0)

        for ch in range(_RCH):
            def _initrow(g16, u, ch=ch):
                dv = dinv_v[pl.ds(ch * _CH + g16 * 16, 16)]
                for r in range(16):
                    row = g16 * 16 + r
                    av = lax.broadcast(dv[r], (16,))
                    for g in range(_F // 16):
                        sl = pl.ds(g * 16, 16)
                        ubuf[row, sl] = av * h0_v[ch * _CH + row, sl]
                return u
            lax.fori_loop(0, _CH // 16, _initrow, 0)
            rows = pl.ds(rbase + ch * _CH, _CH)
            pltpu.sync_copy(ubuf, Z.at[rows])
            pltpu.sync_copy(ubuf, S.at[rows])

        def _edge_pass():
            plsc.subcore_barrier()
            for p in range(8):
                pltpu.async_copy(Z.at[src_v.at[p]], gbuf.at[p], gsem)

            def _eb(jj, u):
                for p in range(10):
                    j = jj * 10 + p
                    pltpu.make_async_copy(
                        Z.at[src_v.at[j]], gbuf.at[p], gsem).wait()
                    pltpu.async_copy(gbuf.at[p], S.at[dst_v.at[j]], ssem, add=True)
                    @pl.when(j + 8 < _NCHUNK)
                    def _():
                        @pl.when(j >= 2)
                        def _():
                            pltpu.make_async_copy(
                                gbuf.at[(p + 8) % 10],
                                S.at[dst_v.at[j - 2]], ssem).wait()
                        pltpu.async_copy(
                            Z.at[src_v.at[j + 8]], gbuf.at[(p + 8) % 10], gsem)
                return u
            lax.fori_loop(0, _NCHUNK // 10, _eb, 0)
            for p in range(10):
                pltpu.make_async_copy(
                    gbuf.at[(_NCHUNK - 10 + p) % 10],
                    S.at[dst_v.at[_NCHUNK - 10 + p]], ssem).wait()
            plsc.subcore_barrier()

        def _update(write_out):
            def _wtarget(ch):
                rows = pl.ds(rbase + ch * _CH, _CH)
                if write_out:
                    return out_ref.at[rows, pl.ds(fbase, _F)]
                return Z.at[rows]

            for ch in range(_RCH):
                rows = pl.ds(rbase + ch * _CH, _CH)
                b = ubuf if ch % 2 == 0 else gbuf.at[1]
                pltpu.sync_copy(S.at[rows], gbuf.at[0])
                if ch >= 2:
                    pltpu.make_async_copy(b, _wtarget(ch - 2), gsem).wait()

                def _ur(g16, u, ch=ch, b=b):
                    dv = dinv_v[pl.ds(ch * _CH + g16 * 16, 16)]
                    for r in range(16):
                        row = g16 * 16 + r
                        av = lax.broadcast(dv[r], (16,))
                        for g in range(_F // 16):
                            sl = pl.ds(g * 16, 16)
                            z = ((1.0 - _ALPHA) * (av * gbuf[0, row, sl])
                                 + _ALPHA * h0_v[ch * _CH + row, sl])
                            b[row, sl] = z if write_out else av * z
                    return u
                lax.fori_loop(0, _CH // 16, _ur, 0)
                pltpu.async_copy(b, _wtarget(ch), gsem)
                if not write_out:
                    pltpu.sync_copy(b, S.at[rows])
            for ch in range(_RCH - 2, _RCH):
                b = ubuf if ch % 2 == 0 else gbuf.at[1]
                pltpu.make_async_copy(b, _wtarget(ch), gsem).wait()

        def _ib(k, u):
            _edge_pass()
            _update(False)
            return u
        lax.fori_loop(0, _K - 1, _ib, 0)
        _edge_pass()
        _update(True)

    return prop


_prop = _make_prop()


def kernel(x, edge_index, W1, b1, W2, b2):
    z0 = _mlp(x, W1, b1, W2, b2)

    pad_e = _EPAD - _E
    pad_src = (jnp.arange(pad_e, dtype=jnp.int32) * 37) % _N
    pad_dst = _N + (jnp.arange(pad_e, dtype=jnp.int32) % (_NPAD - _N))
    pad_pair = jnp.stack([pad_src, pad_dst])
    edges = jnp.concatenate([edge_index, pad_pair], axis=1)
    edges = edges.reshape(2, _NT, _NCHUNK, _CH)

    zfin = _prop(z0, edges)
    return _lsm(zfin)
